# baseline (speedup 1.0000x reference)
import jax
import jax.numpy as jnp
from jax.experimental import pallas as pl


_TS = 512


def _add_kernel(x_ref, p_ref, o_ref):
    o_ref[...] = x_ref[...] + p_ref[...][None, :, :]


def kernel(x, pos_table):
    B, S, D = x.shape
    ts = _TS
    return pl.pallas_call(
        _add_kernel,
        grid=(S // ts,),
        in_specs=[
            pl.BlockSpec((B, ts, D), lambda s: (0, s, 0)),
            pl.BlockSpec((ts, D), lambda s: (s, 0)),
        ],
        out_specs=pl.BlockSpec((B, ts, D), lambda s: (0, s, 0)),
        out_shape=jax.ShapeDtypeStruct((B, S, D), x.dtype),
    )(x, pos_table)
